# baseline (speedup 1.0000x reference)
import functools

import jax
import jax.numpy as jnp
from jax import lax
from jax.experimental import pallas as pl
from jax.experimental.pallas import tpu as pltpu
from jax.experimental.pallas import tpu_sc as plsc

_CHUNK = 256
_L = 16


@functools.cache
def _make_gather_kernel(V, D, B):
    info = plsc.get_sparse_core_info()
    NC, NS = info.num_cores, info.num_subcores
    NW = NC * NS
    assert B % NW == 0
    b_per_w = B // NW
    n_chunks = b_per_w // _CHUNK
    assert b_per_w % _CHUNK == 0
    rows_per_line = 128 // D
    mesh = plsc.VectorSubcoreMesh(core_axis_name="c", subcore_axis_name="s")

    out_sds = jax.ShapeDtypeStruct((B, D), jnp.float32)

    @functools.partial(
        pl.kernel,
        mesh=mesh,
        out_type=(out_sds, out_sds),
        scratch_types=[
            pltpu.VMEM((b_per_w,), jnp.int32),
            pltpu.VMEM((b_per_w,), jnp.int32),
            pltpu.VMEM((_CHUNK,), jnp.int32),
            pltpu.VMEM((_CHUNK,), jnp.int32),
            pltpu.VMEM((_CHUNK, 128), jnp.float32),
            pltpu.VMEM((_CHUNK, 128), jnp.float32),
            pltpu.VMEM((_CHUNK, D), jnp.float32),
            pltpu.SemaphoreType.DMA,
            pltpu.SemaphoreType.DMA,
        ],
        compiler_params=pltpu.CompilerParams(
            use_tc_tiling_on_sc=True, needs_layout_passes=False
        ),
    )
    def k(tab_l, tab_r, idx_l, idx_r, out_l, out_r,
          vidx_l, vidx_r, gidx_a, gidx_b, lines_a, lines_b, out_v,
          sem_a, sem_b):
        wid = lax.axis_index("s") * NC + lax.axis_index("c")
        base = wid * b_per_w
        pltpu.sync_copy(idx_l.at[pl.ds(base, b_per_w)], vidx_l)
        pltpu.sync_copy(idx_r.at[pl.ds(base, b_per_w)], vidx_r)

        lanes = lax.iota(jnp.int32, _L)

        def line_ids(vidx, gidx, c0):
            def body(i, carry):
                v = vidx[pl.ds(c0 + i * _L, _L)]
                gidx[pl.ds(i * _L, _L)] = lax.shift_right_logical(
                    v, jnp.int32(2)
                )
                return carry
            lax.fori_loop(0, _CHUNK // _L, body, 0)

        def start_gather(tab, gidx, lines_v, sem):
            return pltpu.make_async_copy(tab.at[gidx], lines_v, sem)

        def extract(vidx, lines_v, c0):
            def body(i, carry):
                rows = i * _L + lanes
                sub = lax.bitwise_and(vidx[pl.ds(c0 + i * _L, _L)],
                                      jnp.int32(rows_per_line - 1))
                colbase = sub * jnp.int32(D)
                for c in range(D):
                    vals = plsc.load_gather(lines_v, [rows, colbase + c])
                    plsc.store_scatter(
                        out_v, [rows, jnp.full((_L,), c, jnp.int32)], vals
                    )
                return carry
            lax.fori_loop(0, _CHUNK // _L, body, 0)

        for tab, vidx, out in ((tab_l, vidx_l, out_l), (tab_r, vidx_r, out_r)):
            for c in range(n_chunks):
                gidx, lines_v, sem = (
                    (gidx_a, lines_a, sem_a) if c % 2 == 0
                    else (gidx_b, lines_b, sem_b)
                )
                line_ids(vidx, gidx, c * _CHUNK)
                start_gather(tab, gidx, lines_v, sem).start()
                start_gather(tab, gidx, lines_v, sem).wait()
                extract(vidx, lines_v, c * _CHUNK)
                pltpu.sync_copy(
                    out_v, out.at[pl.ds(base + c * _CHUNK, _CHUNK)]
                )

    return k


def kernel(table_left, table_right, indices_left, indices_right):
    V, D = table_left.shape
    (B,) = indices_left.shape
    k = _make_gather_kernel(V, D, B)
    tab_l = table_left.reshape(V * D // 128, 128)
    tab_r = table_right.reshape(V * D // 128, 128)
    return k(
        tab_l,
        tab_r,
        indices_left.astype(jnp.int32),
        indices_right.astype(jnp.int32),
    )


# speedup vs baseline: 1.5480x; 1.5480x over previous
import functools

import jax
import jax.numpy as jnp
from jax import lax
from jax.experimental import pallas as pl
from jax.experimental.pallas import tpu as pltpu
from jax.experimental.pallas import tpu_sc as plsc

_CHUNK = 256
_FIRE = 16
_NSEM = 4


@functools.cache
def _make_gather_kernel(V, D, B):
    info = plsc.get_sparse_core_info()
    NC, NS = info.num_cores, info.num_subcores
    NW = NC * NS
    assert B % NW == 0
    b_per_w = B // NW
    n_chunks = b_per_w // _CHUNK
    assert b_per_w % _CHUNK == 0 and _CHUNK % (_FIRE * _NSEM) == 0
    mesh = plsc.VectorSubcoreMesh(core_axis_name="c", subcore_axis_name="s")

    out_sds = jax.ShapeDtypeStruct((B, D), jnp.float32)

    @functools.partial(
        pl.kernel,
        mesh=mesh,
        out_type=(out_sds, out_sds),
        scratch_types=[
            pltpu.VMEM((b_per_w,), jnp.int32),
            pltpu.VMEM((b_per_w,), jnp.int32),
            pltpu.VMEM((_CHUNK, D), jnp.float32),
            pltpu.VMEM((_CHUNK, D), jnp.float32),
            [pltpu.SemaphoreType.DMA] * _NSEM,
            [pltpu.SemaphoreType.DMA] * _NSEM,
        ],
        compiler_params=pltpu.CompilerParams(use_tc_tiling_on_sc=True),
    )
    def k(tab_l, tab_r, idx_l, idx_r, out_l, out_r,
          vidx_l, vidx_r, rows_a, rows_b, sems_a, sems_b):
        wid = lax.axis_index("s") * NC + lax.axis_index("c")
        base = wid * b_per_w
        pltpu.sync_copy(idx_l.at[pl.ds(base, b_per_w)], vidx_l)
        pltpu.sync_copy(idx_r.at[pl.ds(base, b_per_w)], vidx_r)

        def fire_chunk(tab, vidx, rows_v, sems, c0):
            def body(i, carry):
                r0 = i * _FIRE
                v = vidx[pl.ds(c0 + r0, _FIRE)]
                for j in range(_FIRE):
                    s = v[j]
                    pltpu.make_async_copy(
                        tab.at[pl.ds(s, 1)],
                        rows_v.at[pl.ds(r0 + j, 1)],
                        sems[j % _NSEM],
                    ).start()
                return carry
            lax.fori_loop(0, _CHUNK // _FIRE, body, 0)

        def drain_chunk(tab, rows_v, sems):
            def body(i, carry):
                for q in range(_NSEM):
                    pltpu.make_async_copy(
                        tab.at[pl.ds(0, 1)], rows_v.at[pl.ds(0, 1)], sems[q]
                    ).wait()
                return carry
            lax.fori_loop(0, _CHUNK // _NSEM, body, 0)

        for tab, vidx, out in ((tab_l, vidx_l, out_l), (tab_r, vidx_r, out_r)):
            for c in range(n_chunks):
                rows_v, sems = (rows_a, sems_a) if c % 2 == 0 else (rows_b, sems_b)
                fire_chunk(tab, vidx, rows_v, sems, c * _CHUNK)
                drain_chunk(tab, rows_v, sems)
                pltpu.sync_copy(
                    rows_v, out.at[pl.ds(base + c * _CHUNK, _CHUNK)]
                )

    return k


def kernel(table_left, table_right, indices_left, indices_right):
    V, D = table_left.shape
    (B,) = indices_left.shape
    k = _make_gather_kernel(V, D, B)
    return k(
        table_left,
        table_right,
        indices_left.astype(jnp.int32),
        indices_right.astype(jnp.int32),
    )
